# baseline (speedup 1.0000x reference)
import jax
import jax.numpy as jnp
from jax.experimental import pallas as pl
from jax.experimental.pallas import tpu as pltpu

_GH, _GW, _D = 32, 32, 768
_N = _GH * _GW
_NQ = 8
_CH = (_N + 1) // _NQ


def _pos_emb_body(row_ref, col_ref, cls_ref, out_ref, buf_ref, sems):
    i = jax.lax.broadcasted_iota(jnp.int32, (_N + 1, _GH), 0)
    j = jax.lax.broadcasted_iota(jnp.int32, (_N + 1, _GH), 1)
    cell = i - 1
    sel_row = ((cell // _GW) == j).astype(jnp.float32)
    sel_col = (((cell % _GW) == j) & (cell >= 0)).astype(jnp.float32)
    left = jnp.dot(sel_row, row_ref[...], preferred_element_type=jnp.float32)
    right = jnp.dot(sel_col, col_ref[...], preferred_element_type=jnp.float32)
    rows = jnp.concatenate([left, right], axis=1)
    i2 = jax.lax.broadcasted_iota(jnp.int32, (_N + 1, _D), 0)
    buf_ref[...] = jnp.where(i2 == 0, cls_ref[...], rows)

    copies = []
    for q in range(_NQ):
        lo = q * _CH
        n = _CH if q < _NQ - 1 else (_N + 1 - lo)
        cp = pltpu.make_async_copy(
            buf_ref.at[pl.ds(lo, n)], out_ref.at[0, pl.ds(lo, n)], sems.at[q]
        )
        cp.start()
        copies.append(cp)
    for cp in copies:
        cp.wait()


@jax.jit
def kernel(row_W, col_W, cls_pos):
    cls2d = cls_pos.reshape(1, _D)
    return pl.pallas_call(
        _pos_emb_body,
        out_specs=pl.BlockSpec(memory_space=pl.ANY),
        out_shape=jax.ShapeDtypeStruct((1, _N + 1, _D), jnp.float32),
        scratch_shapes=[
            pltpu.VMEM((_N + 1, _D), jnp.float32),
            pltpu.SemaphoreType.DMA((_NQ,)),
        ],
    )(row_W, col_W, cls2d)


# speedup vs baseline: 3.8403x; 3.8403x over previous
import jax
import jax.numpy as jnp
from jax.experimental import pallas as pl
from jax.experimental.pallas import tpu as pltpu

_GH, _GW, _D = 32, 32, 768
_N = _GH * _GW
_NQ = 8
_CH = (_N + 1) // _NQ


def _pos_emb_body(row_ref, col_ref, cls_ref, out_ref, buf_ref, sems):
    i = jax.lax.broadcasted_iota(jnp.int32, (_N + 1, _GH), 0)
    j = jax.lax.broadcasted_iota(jnp.int32, (_N + 1, _GH), 1)
    cell = i - 1
    sel_row = ((cell // _GW) == j).astype(jnp.float32)
    sel_col = (((cell % _GW) == j) & (cell >= 0)).astype(jnp.float32)
    left = jnp.dot(sel_row, row_ref[...], preferred_element_type=jnp.float32)
    right = jnp.dot(sel_col, col_ref[...], preferred_element_type=jnp.float32)
    rows = jnp.concatenate([left, right], axis=1)
    i2 = jax.lax.broadcasted_iota(jnp.int32, (_N + 1, _D), 0)
    buf_ref[...] = jnp.where(i2 == 0, cls_ref[...], rows)

    copies = []
    for q in range(_NQ):
        lo = q * _CH
        n = _CH if q < _NQ - 1 else (_N + 1 - lo)
        cp = pltpu.make_async_copy(
            buf_ref.at[pl.ds(lo, n)], out_ref.at[pl.ds(lo, n), 0], sems.at[q]
        )
        cp.start()
        copies.append(cp)
    for cp in copies:
        cp.wait()


@jax.jit
def kernel(row_W, col_W, cls_pos):
    cls2d = cls_pos.reshape(1, _D)
    out = pl.pallas_call(
        _pos_emb_body,
        out_specs=pl.BlockSpec(memory_space=pl.ANY),
        out_shape=jax.ShapeDtypeStruct((_N + 1, 1, _D), jnp.float32),
        scratch_shapes=[
            pltpu.VMEM((_N + 1, _D), jnp.float32),
            pltpu.SemaphoreType.DMA((_NQ,)),
        ],
    )(row_W, col_W, cls2d)
    return out.reshape(1, _N + 1, _D)


# speedup vs baseline: 4.5398x; 1.1822x over previous
import jax
import jax.numpy as jnp
from jax.experimental import pallas as pl
from jax.experimental.pallas import tpu as pltpu

_GH, _GW, _D = 32, 32, 768
_N = _GH * _GW
_NQ = 8
_CH = (_N + 1) // _NQ


def _pos_emb_body(row_ref, col_ref, cls_ref, out_ref, buf_ref, sems):
    row_w = row_ref[...]
    col_w = col_ref[...]
    copies = []
    for q in range(_NQ):
        lo = q * _CH
        n = _CH if q < _NQ - 1 else (_N + 1 - lo)
        i = jax.lax.broadcasted_iota(jnp.int32, (n, _GH), 0) + lo
        j = jax.lax.broadcasted_iota(jnp.int32, (n, _GH), 1)
        cell = i - 1
        sel_row = ((cell // _GW) == j).astype(jnp.float32)
        sel_col = (((cell % _GW) == j) & (cell >= 0)).astype(jnp.float32)
        left = jnp.dot(sel_row, row_w, preferred_element_type=jnp.float32)
        right = jnp.dot(sel_col, col_w, preferred_element_type=jnp.float32)
        rows = jnp.concatenate([left, right], axis=1)
        if q == 0:
            i2 = jax.lax.broadcasted_iota(jnp.int32, (n, _D), 0)
            rows = jnp.where(i2 == 0, cls_ref[...], rows)
        buf_ref[pl.ds(lo, n), :] = rows
        cp = pltpu.make_async_copy(
            buf_ref.at[pl.ds(lo, n)], out_ref.at[pl.ds(lo, n), 0], sems.at[q]
        )
        cp.start()
        copies.append(cp)
    for cp in copies:
        cp.wait()


@jax.jit
def kernel(row_W, col_W, cls_pos):
    cls2d = cls_pos.reshape(1, _D)
    out = pl.pallas_call(
        _pos_emb_body,
        out_specs=pl.BlockSpec(memory_space=pl.ANY),
        out_shape=jax.ShapeDtypeStruct((_N + 1, 1, _D), jnp.float32),
        scratch_shapes=[
            pltpu.VMEM((_N + 1, _D), jnp.float32),
            pltpu.SemaphoreType.DMA((_NQ,)),
        ],
    )(row_W, col_W, cls2d)
    return out.reshape(1, _N + 1, _D)
